# baseline (speedup 1.0000x reference)
import functools

import jax
import jax.numpy as jnp
from jax import lax
from jax.experimental import pallas as pl
from jax.experimental.pallas import tpu as pltpu
from jax.experimental.pallas import tpu_sc as plsc

VOCAB = 1000000
EMB = 64
BATCH = 16384
NEG = 10

NC = 2
NS = 16
NW = NC * NS
ROWS_W = BATCH // NW
NEG_W = ROWS_W * NEG
ITEMS_W = 2 * ROWS_W + NEG_W
NROW = ITEMS_W // 128


def _sc_scores_body(center_hbm, pos_hbm, neg_hbm, in_hbm, out_hbm,
                    sp_out, sn_out,
                    idx0, idx1, bk, val0, val1, acc_p, acc_n, sem0, sem1):
    wid = lax.axis_index("s") * NC + lax.axis_index("c")
    vals = (val0, val1)
    idxs = (idx0, idx1)
    sems = (sem0, sem1)
    iota16 = lax.broadcasted_iota(jnp.int32, (16,), 0)
    base = wid * ROWS_W

    for k in range(NROW):
        if k < 4:
            src = center_hbm.at[pl.ds(base + k * 128, 128)]
        elif k < 8:
            src = pos_hbm.at[pl.ds(base + (k - 4) * 128, 128)]
        else:
            src = neg_hbm.at[pl.ds(base * NEG + (k - 8) * 128, 128)]
        pltpu.sync_copy(src, idx0.at[k])

    def init_aux(g, _):
        sl = pl.ds(g * 16, 16)
        kvec = g * 16 + iota16
        bk[sl] = kvec // NEG
        acc_n[sl] = jnp.zeros((16,), jnp.float32)
        return 0
    lax.fori_loop(0, NEG_W // 16, init_aux, 0, unroll=8)

    def init_acc_p(g, _):
        acc_p[pl.ds(g * 16, 16)] = jnp.zeros((16,), jnp.float32)
        return 0

    def init_idx1(r, _):
        for u in range(8):
            sl = pl.ds(u * 16, 16)
            idx1[r, sl] = idx0[r, sl] + VOCAB
        return 0

    def adv(ref):
        def body(r, _):
            for u in range(8):
                sl = pl.ds(u * 16, 16)
                ref[r, sl] = ref[r, sl] + 2 * VOCAB
            return 0
        lax.fori_loop(0, NROW, body, 0, unroll=4)

    def issue(s):
        for k in range(NROW):
            tbl = in_hbm if k < 4 else out_hbm
            pltpu.async_copy(tbl.at[idxs[s].at[k]],
                             vals[s].at[pl.ds(k * 128, 128)], sems[s])

    def drain(s):
        pltpu.make_async_copy(
            sn_out.at[pl.ds(0, ITEMS_W)], vals[s], sems[s]).wait()

    def compute(s):
        val = vals[s]

        def pos_body(g, _):
            sl = pl.ds(g * 16, 16)
            acc_p[sl] = acc_p[sl] + val[sl] * val[pl.ds(ROWS_W + g * 16, 16)]
            return 0
        lax.fori_loop(0, ROWS_W // 16, pos_body, 0, unroll=8)

        def neg_body(g, _):
            sl = pl.ds(g * 16, 16)
            vv = plsc.load_gather(val, [bk[sl]])
            un = val[pl.ds(2 * ROWS_W + g * 16, 16)]
            acc_n[sl] = acc_n[sl] + vv * un
            return 0
        lax.fori_loop(0, NEG_W // 16, neg_body, 0, unroll=8)

    lax.fori_loop(0, ROWS_W // 16, init_acc_p, 0, unroll=8)
    lax.fori_loop(0, NROW, init_idx1, 0, unroll=4)
    issue(0)

    def step(t, _):
        @pl.when(t > 0)
        def _():
            adv(idx1)
        issue(1)
        drain(0)
        compute(0)

        @pl.when(t < EMB // 2 - 1)
        def _():
            adv(idx0)
            issue(0)
        drain(1)
        compute(1)
        return 0

    lax.fori_loop(0, EMB // 2, step, 0)

    pltpu.sync_copy(acc_p, sp_out.at[pl.ds(wid * ROWS_W, ROWS_W)])
    pltpu.sync_copy(acc_n, sn_out.at[pl.ds(wid * NEG_W, NEG_W)])


def _loss_body(sp_ref, sn_ref, out_ref):
    ps = sp_ref[...]
    ns = sn_ref[...]
    pls = jnp.minimum(ps, 0.0) - jnp.log1p(jnp.exp(-jnp.abs(ps)))
    nls = jnp.minimum(-ns, 0.0) - jnp.log1p(jnp.exp(-jnp.abs(ns)))
    out_ref[0, 0] = -(jnp.sum(pls) / BATCH) - (jnp.sum(nls) / (BATCH * NEG))


@jax.jit
def kernel(in_embed, out_embed, center, pos, neg):
    center = center.astype(jnp.int32)
    pos = pos.astype(jnp.int32)
    neg_flat = jnp.reshape(neg.astype(jnp.int32), (BATCH * NEG,))
    in_flat = jnp.reshape(in_embed.T, (VOCAB * EMB,))
    out_flat = jnp.reshape(out_embed.T, (VOCAB * EMB,))

    mesh = plsc.VectorSubcoreMesh(core_axis_name="c", subcore_axis_name="s")
    sc_scores = functools.partial(
        pl.kernel,
        mesh=mesh,
        compiler_params=pltpu.CompilerParams(
            needs_layout_passes=False, use_tc_tiling_on_sc=False),
        out_type=[jax.ShapeDtypeStruct((BATCH,), jnp.float32),
                  jax.ShapeDtypeStruct((BATCH * NEG,), jnp.float32)],
        scratch_types=[
            pltpu.VMEM((NROW, 128), jnp.int32),
            pltpu.VMEM((NROW, 128), jnp.int32),
            pltpu.VMEM((NEG_W,), jnp.int32),
            pltpu.VMEM((ITEMS_W,), jnp.float32),
            pltpu.VMEM((ITEMS_W,), jnp.float32),
            pltpu.VMEM((ROWS_W,), jnp.float32),
            pltpu.VMEM((NEG_W,), jnp.float32),
            pltpu.SemaphoreType.DMA,
            pltpu.SemaphoreType.DMA,
        ],
    )(_sc_scores_body)
    sp, sn = sc_scores(center, pos, neg_flat, in_flat, out_flat)

    loss = pl.pallas_call(
        _loss_body,
        out_shape=jax.ShapeDtypeStruct((1, 1), jnp.float32),
        out_specs=pl.BlockSpec(memory_space=pltpu.SMEM),
    )(jnp.reshape(sp, (BATCH // 128, 128)),
      jnp.reshape(sn, (BATCH * NEG // 128, 128)))
    return loss[0, 0]


# speedup vs baseline: 7.7919x; 7.7919x over previous
import functools

import jax
import jax.numpy as jnp
from jax import lax
from jax.experimental import pallas as pl
from jax.experimental.pallas import tpu as pltpu
from jax.experimental.pallas import tpu_sc as plsc

VOCAB = 1000000
EMB = 64
BATCH = 16384
NEG = 10

NC = 2
NS = 16
NW = NC * NS
ROWS_W = BATCH // NW
CHUNK = 32
NCHUNK = ROWS_W // CHUNK
NEG_CH = CHUNK * NEG


def _sc_scores_body(center_hbm, pos_hbm, neg_hbm, in_hbm, out_hbm,
                    sp_out, sn_out,
                    idx_c0, idx_c1, idx_p0, idx_p1, idx_n0, idx_n1,
                    pair_c0, pair_c1, pair_p0, pair_p1, pair_n0, pair_n1,
                    rows_v0, rows_v1, rows_p0, rows_p1, rows_n0, rows_n1,
                    sp, sn, sem0, sem1):
    wid = lax.axis_index("s") * NC + lax.axis_index("c")
    idx_c = (idx_c0, idx_c1)
    idx_p = (idx_p0, idx_p1)
    idx_n = (idx_n0, idx_n1)
    pair_c = (pair_c0, pair_c1)
    pair_p = (pair_p0, pair_p1)
    pair_n = (pair_n0, pair_n1)
    rows_v = (rows_v0, rows_v1)
    rows_p = (rows_p0, rows_p1)
    rows_n = (rows_n0, rows_n1)
    sems = (sem0, sem1)
    iota16 = lax.broadcasted_iota(jnp.int32, (16,), 0)

    def issue(c, s):
        base = wid * ROWS_W + c * CHUNK
        pltpu.sync_copy(center_hbm.at[pl.ds(base, CHUNK)], idx_c[s])
        pltpu.sync_copy(pos_hbm.at[pl.ds(base, CHUNK)], idx_p[s])
        pltpu.sync_copy(neg_hbm.at[pl.ds(base * NEG, NEG_CH)], idx_n[s])
        for g in range(CHUNK // 16):
            sl = pl.ds(g * 16, 16)
            pair_c[s][sl] = lax.shift_right_logical(idx_c[s][sl], 1)
            pair_p[s][sl] = lax.shift_right_logical(idx_p[s][sl], 1)
        for g in range(NEG_CH // 16):
            sl = pl.ds(g * 16, 16)
            pair_n[s][sl] = lax.shift_right_logical(idx_n[s][sl], 1)
        cps = [pltpu.async_copy(in_hbm.at[pair_c[s]], rows_v[s], sems[s]),
               pltpu.async_copy(out_hbm.at[pair_p[s]], rows_p[s], sems[s])]
        for lo, n in ((0, 128), (128, 128), (256, 64)):
            cps.append(pltpu.async_copy(out_hbm.at[pair_n[s].at[pl.ds(lo, n)]],
                                        rows_n[s].at[pl.ds(lo, n)],
                                        sems[s]))
        return cps

    def compute(c, s):
        for g in range(CHUNK // 16):
            sl = pl.ds(g * 16, 16)
            r_idx = g * 16 + iota16
            half_c = lax.bitwise_and(idx_c[s][sl], 1) * EMB
            half_p = lax.bitwise_and(idx_p[s][sl], 1) * EMB
            p_idx = [(g * 16 + iota16) * NEG + j for j in range(NEG)]
            half_n = [lax.bitwise_and(
                plsc.load_gather(idx_n[s], [p_idx[j]]), 1) * EMB
                for j in range(NEG)]
            zeros = jnp.zeros((16,), jnp.float32)

            def body(d, accs):
                col = jnp.broadcast_to(d, (16,))
                vv = plsc.load_gather(rows_v[s], [r_idx, half_c + col])
                up = plsc.load_gather(rows_p[s], [r_idx, half_p + col])
                new = [accs[0] + vv * up]
                for j in range(NEG):
                    un = plsc.load_gather(rows_n[s], [p_idx[j], half_n[j] + col])
                    new.append(accs[1 + j] + vv * un)
                return tuple(new)

            accs = lax.fori_loop(0, EMB, body, (zeros,) * (1 + NEG))
            sp[pl.ds(c * CHUNK + g * 16, 16)] = accs[0]
            for j in range(NEG):
                plsc.store_scatter(
                    sn, [(c * CHUNK + g * 16 + iota16) * NEG + j], accs[1 + j])

    cps = issue(0, 0)
    for c in range(NCHUNK):
        s = c % 2
        nxt = issue(c + 1, 1 - s) if c + 1 < NCHUNK else None
        for cp in cps:
            cp.wait()
        compute(c, s)
        cps = nxt

    pltpu.sync_copy(sp, sp_out.at[pl.ds(wid * ROWS_W, ROWS_W)])
    pltpu.sync_copy(sn, sn_out.at[pl.ds(wid * ROWS_W * NEG, ROWS_W * NEG)])


def _loss_body(sp_ref, sn_ref, out_ref):
    ps = sp_ref[...]
    ns = sn_ref[...]
    pls = jnp.minimum(ps, 0.0) - jnp.log1p(jnp.exp(-jnp.abs(ps)))
    nls = jnp.minimum(-ns, 0.0) - jnp.log1p(jnp.exp(-jnp.abs(ns)))
    out_ref[0, 0] = -(jnp.sum(pls) / BATCH) - (jnp.sum(nls) / (BATCH * NEG))


@jax.jit
def kernel(in_embed, out_embed, center, pos, neg):
    center = center.astype(jnp.int32)
    pos = pos.astype(jnp.int32)
    neg_flat = jnp.reshape(neg.astype(jnp.int32), (BATCH * NEG,))
    in2 = jnp.reshape(in_embed, (VOCAB // 2, 2 * EMB))
    out2 = jnp.reshape(out_embed, (VOCAB // 2, 2 * EMB))

    mesh = plsc.VectorSubcoreMesh(core_axis_name="c", subcore_axis_name="s")
    sc_scores = functools.partial(
        pl.kernel,
        mesh=mesh,
        compiler_params=pltpu.CompilerParams(
            needs_layout_passes=False, use_tc_tiling_on_sc=False),
        out_type=[jax.ShapeDtypeStruct((BATCH,), jnp.float32),
                  jax.ShapeDtypeStruct((BATCH * NEG,), jnp.float32)],
        scratch_types=[
            pltpu.VMEM((CHUNK,), jnp.int32), pltpu.VMEM((CHUNK,), jnp.int32),
            pltpu.VMEM((CHUNK,), jnp.int32), pltpu.VMEM((CHUNK,), jnp.int32),
            pltpu.VMEM((NEG_CH,), jnp.int32), pltpu.VMEM((NEG_CH,), jnp.int32),
            pltpu.VMEM((CHUNK,), jnp.int32), pltpu.VMEM((CHUNK,), jnp.int32),
            pltpu.VMEM((CHUNK,), jnp.int32), pltpu.VMEM((CHUNK,), jnp.int32),
            pltpu.VMEM((NEG_CH,), jnp.int32), pltpu.VMEM((NEG_CH,), jnp.int32),
            pltpu.VMEM((CHUNK, 2 * EMB), jnp.float32),
            pltpu.VMEM((CHUNK, 2 * EMB), jnp.float32),
            pltpu.VMEM((CHUNK, 2 * EMB), jnp.float32),
            pltpu.VMEM((CHUNK, 2 * EMB), jnp.float32),
            pltpu.VMEM((NEG_CH, 2 * EMB), jnp.float32),
            pltpu.VMEM((NEG_CH, 2 * EMB), jnp.float32),
            pltpu.VMEM((ROWS_W,), jnp.float32),
            pltpu.VMEM((ROWS_W * NEG,), jnp.float32),
            pltpu.SemaphoreType.DMA,
            pltpu.SemaphoreType.DMA,
        ],
    )(_sc_scores_body)
    sp, sn = sc_scores(center, pos, neg_flat, in2, out2)

    loss = pl.pallas_call(
        _loss_body,
        out_shape=jax.ShapeDtypeStruct((1, 1), jnp.float32),
        out_specs=pl.BlockSpec(memory_space=pltpu.SMEM),
    )(jnp.reshape(sp, (BATCH // 128, 128)),
      jnp.reshape(sn, (BATCH * NEG // 128, 128)))
    return loss[0, 0]
